# baseline (speedup 1.0000x reference)
import functools

import jax
import jax.numpy as jnp
from jax.experimental import pallas as pl
from jax.experimental.pallas import tpu as pltpu

N = 10000
D = 512
BX = 2000
NX = N // BX
BM = 512
NM = -(-N // BM)


def _gcn_kernel(adj_ref, x_ref, w_ref, out_ref, s_ref):
    i = pl.program_id(0)

    @pl.when(i < NX)
    def _():
        sb = jnp.dot(x_ref[...], w_ref[...], preferred_element_type=jnp.float32)
        s_ref[pl.ds(i * BX, BX), :] = sb.astype(jnp.bfloat16)

    @pl.when(i >= NX)
    def _():
        out_ref[...] = jax.lax.dot_general(
            adj_ref[...], s_ref[...],
            dimension_numbers=(((1,), (0,)), ((), ())),
            preferred_element_type=jnp.float32)


def kernel(x, adj, W):
    return pl.pallas_call(
        _gcn_kernel,
        grid=(NX + NM,),
        in_specs=[
            pl.BlockSpec((BM, N), lambda i: (jnp.maximum(i - NX, 0), 0)),
            pl.BlockSpec((BX, D), lambda i: (jnp.minimum(i, NX - 1), 0)),
            pl.BlockSpec((D, D), lambda i: (0, 0)),
        ],
        out_specs=pl.BlockSpec((BM, D), lambda i: (jnp.maximum(i - NX, 0), 0)),
        out_shape=jax.ShapeDtypeStruct((N, D), jnp.float32),
        scratch_shapes=[pltpu.VMEM((N, D), jnp.bfloat16)],
        compiler_params=pltpu.CompilerParams(
            dimension_semantics=("arbitrary",),
            vmem_limit_bytes=100 * 1024 * 1024,
        ),
    )(adj, x, W)


# speedup vs baseline: 1.0065x; 1.0065x over previous
import functools

import jax
import jax.numpy as jnp
from jax.experimental import pallas as pl
from jax.experimental.pallas import tpu as pltpu

N = 10000
D = 512
BX = 2000
NX = N // BX
BM = 512
NM = -(-N // BM)


def _gcn_kernel(adj_ref, x_ref, w_ref, out_ref, s_ref):
    i = pl.program_id(0)

    @pl.when(i < NX)
    def _():
        sb = jnp.dot(x_ref[...], w_ref[...], preferred_element_type=jnp.float32)
        s_ref[pl.ds(i * BX, BX), :] = sb.astype(jnp.bfloat16)

    @pl.when(i >= NX - 1)
    def _():
        out_ref[...] = jax.lax.dot_general(
            adj_ref[...], s_ref[...],
            dimension_numbers=(((1,), (0,)), ((), ())),
            preferred_element_type=jnp.float32)


def kernel(x, adj, W):
    return pl.pallas_call(
        _gcn_kernel,
        grid=(NX - 1 + NM,),
        in_specs=[
            pl.BlockSpec((BM, N), lambda i: (jnp.maximum(i - (NX - 1), 0), 0)),
            pl.BlockSpec((BX, D), lambda i: (jnp.minimum(i, NX - 1), 0)),
            pl.BlockSpec((D, D), lambda i: (0, 0)),
        ],
        out_specs=pl.BlockSpec((BM, D), lambda i: (jnp.maximum(i - (NX - 1), 0), 0)),
        out_shape=jax.ShapeDtypeStruct((N, D), jnp.float32),
        scratch_shapes=[pltpu.VMEM((N, D), jnp.bfloat16)],
        compiler_params=pltpu.CompilerParams(
            dimension_semantics=("arbitrary",),
            vmem_limit_bytes=100 * 1024 * 1024,
        ),
    )(adj, x, W)


# speedup vs baseline: 1.0080x; 1.0016x over previous
import jax
import jax.numpy as jnp
from jax.experimental import pallas as pl
from jax.experimental.pallas import tpu as pltpu

N = 10000
D = 512
BM = 512
NM = -(-N // BM)


def _gcn_kernel(adj_ref, x_ref, w_ref, out_ref):
    t = jnp.dot(adj_ref[...], x_ref[...], preferred_element_type=jnp.float32)
    out_ref[...] = jnp.dot(t, w_ref[...], preferred_element_type=jnp.float32)


def kernel(x, adj, W):
    return pl.pallas_call(
        _gcn_kernel,
        grid=(NM,),
        in_specs=[
            pl.BlockSpec((BM, N), lambda i: (i, 0)),
            pl.BlockSpec((N, D), lambda i: (0, 0)),
            pl.BlockSpec((D, D), lambda i: (0, 0)),
        ],
        out_specs=pl.BlockSpec((BM, D), lambda i: (i, 0)),
        out_shape=jax.ShapeDtypeStruct((N, D), jnp.float32),
        compiler_params=pltpu.CompilerParams(
            dimension_semantics=("parallel",),
            vmem_limit_bytes=100 * 1024 * 1024,
        ),
    )(adj, x, W)
